# initial kernel scaffold (unmeasured)
import jax
import jax.numpy as jnp
from jax import lax
from jax.experimental import pallas as pl
from jax.experimental.pallas import tpu as pltpu

N_DEV = 32
SQ = 1024
SKV = 1024
HQ = 8
DH = 128
D_MODEL = 1024
WINDOW = 128
SCALE = 0.08838834764831843
CHUNK = SQ // N_DEV
N_STEPS = 2 * (N_DEV - 1)


def _body(x_ref, wq_ref, k_ref, v_ref, wo_ref, out_ref,
          ctx_ref, ring_ref, send_sems, recv_sems):
    my = lax.axis_index("i")
    right = lax.rem(my + 1, N_DEV)

    q = jnp.dot(x_ref[:], wq_ref[:], preferred_element_type=jnp.float32)

    qi = lax.broadcasted_iota(jnp.int32, (SQ, SKV), 0)
    ki = lax.broadcasted_iota(jnp.int32, (SQ, SKV), 1)
    bias = jnp.where(jnp.abs(qi - ki) <= WINDOW, 0.0, -1e9).astype(jnp.float32)

    for h in range(HQ):
        qh = q[:, h * DH:(h + 1) * DH].astype(jnp.bfloat16)
        kh = k_ref[h]
        vh = v_ref[h]
        s = lax.dot_general(qh, kh, (((1,), (1,)), ((), ())),
                            preferred_element_type=jnp.float32)
        s = s * SCALE + bias
        m = jnp.max(s, axis=1, keepdims=True)
        w = jnp.exp(s - m)
        w = w / jnp.sum(w, axis=1, keepdims=True)
        ctx_ref[:, h * DH:(h + 1) * DH] = jnp.dot(
            w.astype(jnp.bfloat16), vh,
            preferred_element_type=jnp.float32).astype(jnp.bfloat16)

    out_ref[:] = jnp.dot(ctx_ref[:], wo_ref[:],
                         preferred_element_type=jnp.float32)

    def modp(v):
        return lax.rem(v + 2 * N_DEV, N_DEV)

    for s in range(N_DEV - 1):
        g = s
        send_c = modp(my - s)
        rdma = pltpu.make_async_remote_copy(
            src_ref=out_ref.at[pl.ds(send_c * CHUNK, CHUNK), :],
            dst_ref=ring_ref.at[g],
            send_sem=send_sems.at[g],
            recv_sem=recv_sems.at[g],
            device_id=(right,),
            device_id_type=pl.DeviceIdType.MESH,
        )
        rdma.start()
        rdma.wait()
        recv_c = modp(my - s - 1)
        idx = pl.ds(recv_c * CHUNK, CHUNK)
        out_ref[idx, :] = out_ref[idx, :] + ring_ref[g]

    for s in range(N_DEV - 1):
        g = (N_DEV - 1) + s
        send_c = modp(my + 1 - s)
        rdma = pltpu.make_async_remote_copy(
            src_ref=out_ref.at[pl.ds(send_c * CHUNK, CHUNK), :],
            dst_ref=ring_ref.at[g],
            send_sem=send_sems.at[g],
            recv_sem=recv_sems.at[g],
            device_id=(right,),
            device_id_type=pl.DeviceIdType.MESH,
        )
        rdma.start()
        rdma.wait()
        recv_c = modp(my - s)
        out_ref[pl.ds(recv_c * CHUNK, CHUNK), :] = ring_ref[g]


def kernel(x, Wq, K_ext, V_ext, Wo):
    i = lax.axis_index("i")
    wq_i = lax.dynamic_slice(Wq, (0, i * HQ * DH), (D_MODEL, HQ * DH))
    wo_i = lax.dynamic_slice(Wo, (i * HQ * DH, 0), (HQ * DH, D_MODEL))

    xb = x[0].astype(jnp.bfloat16)
    wq_b = wq_i.astype(jnp.bfloat16)
    wo_b = wo_i.astype(jnp.bfloat16)
    kb = jnp.transpose(K_ext[0], (1, 0, 2)).astype(jnp.bfloat16)
    vb = jnp.transpose(V_ext[0], (1, 0, 2)).astype(jnp.bfloat16)

    out = pl.pallas_call(
        _body,
        out_shape=jax.ShapeDtypeStruct((SQ, D_MODEL), jnp.float32),
        in_specs=[pl.BlockSpec(memory_space=pltpu.VMEM)] * 5,
        out_specs=pl.BlockSpec(memory_space=pltpu.VMEM),
        scratch_shapes=[
            pltpu.VMEM((SQ, HQ * DH), jnp.bfloat16),
            pltpu.VMEM((N_STEPS, CHUNK, D_MODEL), jnp.float32),
            pltpu.SemaphoreType.DMA((N_STEPS,)),
            pltpu.SemaphoreType.DMA((N_STEPS,)),
        ],
        compiler_params=pltpu.CompilerParams(collective_id=0),
    )(xb, wq_b, kb, vb, wo_b)
    return out[None]


# baseline (device time: 249415 ns/iter reference)
import jax
import jax.numpy as jnp
from jax import lax
from jax.experimental import pallas as pl
from jax.experimental.pallas import tpu as pltpu

N_DEV = 32
SQ = 1024
SKV = 1024
HQ = 8
DH = 128
D_MODEL = 1024
WINDOW = 128
SCALE = 0.08838834764831843
CHUNK = SQ // N_DEV
N_STEPS = 2 * (N_DEV - 1)


def _body(x_ref, wq_ref, k_ref, v_ref, wo_ref, out_ref,
          ctx_ref, ring_ref, send_sems, recv_sems):
    my = lax.axis_index("i")
    right = lax.rem(my + 1, N_DEV)

    q = jnp.dot(x_ref[:], wq_ref[:], preferred_element_type=jnp.float32)

    qi = lax.broadcasted_iota(jnp.int32, (SQ, SKV), 0)
    ki = lax.broadcasted_iota(jnp.int32, (SQ, SKV), 1)
    bias = jnp.where(jnp.abs(qi - ki) <= WINDOW, 0.0, -1e9).astype(jnp.float32)

    for h in range(HQ):
        qh = q[:, h * DH:(h + 1) * DH].astype(jnp.bfloat16)
        kh = k_ref[h]
        vh = v_ref[h]
        s = lax.dot_general(qh, kh, (((1,), (1,)), ((), ())),
                            preferred_element_type=jnp.float32)
        s = s * SCALE + bias
        m = jnp.max(s, axis=1, keepdims=True)
        w = jnp.exp(s - m)
        w = w / jnp.sum(w, axis=1, keepdims=True)
        ctx_ref[:, h * DH:(h + 1) * DH] = jnp.dot(
            w.astype(jnp.bfloat16), vh,
            preferred_element_type=jnp.float32).astype(jnp.bfloat16)

    out_ref[:] = jnp.dot(ctx_ref[:], wo_ref[:],
                         preferred_element_type=jnp.float32)

    def modp(v):
        return lax.rem(v + 2 * N_DEV, N_DEV)

    for s in range(N_DEV - 1):
        g = s
        send_c = modp(my - s)
        rdma = pltpu.make_async_remote_copy(
            src_ref=out_ref.at[pl.ds(send_c * CHUNK, CHUNK), :],
            dst_ref=ring_ref.at[g],
            send_sem=send_sems.at[g],
            recv_sem=recv_sems.at[g],
            device_id=(right,),
            device_id_type=pl.DeviceIdType.MESH,
        )
        rdma.start()
        rdma.wait()
        recv_c = modp(my - s - 1)
        idx = pl.ds(recv_c * CHUNK, CHUNK)
        out_ref[idx, :] = out_ref[idx, :] + ring_ref[g]

    for s in range(N_DEV - 1):
        g = (N_DEV - 1) + s
        send_c = modp(my + 1 - s)
        rdma = pltpu.make_async_remote_copy(
            src_ref=out_ref.at[pl.ds(send_c * CHUNK, CHUNK), :],
            dst_ref=ring_ref.at[g],
            send_sem=send_sems.at[g],
            recv_sem=recv_sems.at[g],
            device_id=(right,),
            device_id_type=pl.DeviceIdType.MESH,
        )
        rdma.start()
        rdma.wait()
        recv_c = modp(my - s)
        out_ref[pl.ds(recv_c * CHUNK, CHUNK), :] = ring_ref[g]


def kernel(x, Wq, K_ext, V_ext, Wo):
    i = lax.axis_index("i")
    wq_i = lax.dynamic_slice(Wq, (0, i * HQ * DH), (D_MODEL, HQ * DH))
    wo_i = lax.dynamic_slice(Wo, (i * HQ * DH, 0), (HQ * DH, D_MODEL))

    xb = x[0].astype(jnp.bfloat16)
    wq_b = wq_i.astype(jnp.bfloat16)
    wo_b = wo_i.astype(jnp.bfloat16)
    kb = jnp.transpose(K_ext[0], (1, 0, 2)).astype(jnp.bfloat16)
    vb = jnp.transpose(V_ext[0], (1, 0, 2)).astype(jnp.bfloat16)

    out = pl.pallas_call(
        _body,
        out_shape=jax.ShapeDtypeStruct((SQ, D_MODEL), jnp.float32),
        in_specs=[pl.BlockSpec(memory_space=pltpu.VMEM)] * 5,
        out_specs=pl.BlockSpec(memory_space=pltpu.VMEM),
        scratch_shapes=[
            pltpu.VMEM((SQ, HQ * DH), jnp.bfloat16),
            pltpu.VMEM((N_STEPS, CHUNK, D_MODEL), jnp.float32),
            pltpu.SemaphoreType.DMA((N_STEPS,)),
            pltpu.SemaphoreType.DMA((N_STEPS,)),
        ],
    )(xb, wq_b, kb, vb, wo_b)
    return out[None]


# device time: 101929 ns/iter; 2.4469x vs baseline; 2.4469x over previous
import jax
import jax.numpy as jnp
from jax import lax
from jax.experimental import pallas as pl
from jax.experimental.pallas import tpu as pltpu

N_DEV = 32
SQ = 1024
SKV = 1024
HQ = 8
DH = 128
D_MODEL = 1024
WINDOW = 128
SCALE = 0.08838834764831843
CHUNK = SQ // N_DEV


def _body(x_ref, wq_ref, k_ref, v_ref, wo_ref, out_ref,
          ctx_ref, part_ref, rs_buf,
          rs_send_sems, rs_recv_sems, ag_send_sems, ag_recv_sems):
    my = lax.axis_index("i")

    q = jnp.dot(x_ref[:], wq_ref[:], preferred_element_type=jnp.float32)

    qi = lax.broadcasted_iota(jnp.int32, (SQ, SKV), 0)
    ki = lax.broadcasted_iota(jnp.int32, (SQ, SKV), 1)
    bias = jnp.where(jnp.abs(qi - ki) <= WINDOW, 0.0, -1e9).astype(jnp.float32)

    for h in range(HQ):
        qh = q[:, h * DH:(h + 1) * DH].astype(jnp.bfloat16)
        kh = k_ref[h]
        vh = v_ref[h]
        s = lax.dot_general(qh, kh, (((1,), (1,)), ((), ())),
                            preferred_element_type=jnp.float32)
        s = s * SCALE + bias
        m = jnp.max(s, axis=1, keepdims=True)
        w = jnp.exp(s - m)
        w = w / jnp.sum(w, axis=1, keepdims=True)
        ctx_ref[:, h * DH:(h + 1) * DH] = jnp.dot(
            w.astype(jnp.bfloat16), vh,
            preferred_element_type=jnp.float32).astype(jnp.bfloat16)

    part_ref[:] = jnp.dot(ctx_ref[:], wo_ref[:],
                          preferred_element_type=jnp.float32
                          ).astype(jnp.bfloat16)

    rs_sends = []
    for t in range(N_DEV):
        desc = pltpu.make_async_remote_copy(
            src_ref=part_ref.at[pl.ds(t * CHUNK, CHUNK), :],
            dst_ref=rs_buf.at[my],
            send_sem=rs_send_sems.at[t],
            recv_sem=rs_recv_sems.at[my],
            device_id=(t,),
            device_id_type=pl.DeviceIdType.MESH,
        )
        rs_sends.append(desc)

        @pl.when(t != my)
        def _(desc=desc):
            desc.start()

    rs_buf[my, :, :] = part_ref[pl.ds(my * CHUNK, CHUNK), :]

    for d in range(N_DEV):
        desc = pltpu.make_async_remote_copy(
            src_ref=part_ref.at[pl.ds(0, CHUNK), :],
            dst_ref=rs_buf.at[d],
            send_sem=rs_send_sems.at[d],
            recv_sem=rs_recv_sems.at[d],
            device_id=(0,),
            device_id_type=pl.DeviceIdType.MESH,
        )

        @pl.when(d != my)
        def _(desc=desc):
            desc.wait_recv()

    acc = jnp.sum(rs_buf[:].astype(jnp.float32), axis=0)
    my_rows = pl.ds(my * CHUNK, CHUNK)
    out_ref[my_rows, :] = acc.astype(jnp.bfloat16)

    ag_sends = []
    for t in range(N_DEV):
        desc = pltpu.make_async_remote_copy(
            src_ref=out_ref.at[my_rows, :],
            dst_ref=out_ref.at[my_rows, :],
            send_sem=ag_send_sems.at[t],
            recv_sem=ag_recv_sems.at[my],
            device_id=(t,),
            device_id_type=pl.DeviceIdType.MESH,
        )
        ag_sends.append(desc)

        @pl.when(t != my)
        def _(desc=desc):
            desc.start()

    for d in range(N_DEV):
        desc = pltpu.make_async_remote_copy(
            src_ref=out_ref.at[my_rows, :],
            dst_ref=out_ref.at[pl.ds(d * CHUNK, CHUNK), :],
            send_sem=ag_send_sems.at[d],
            recv_sem=ag_recv_sems.at[d],
            device_id=(0,),
            device_id_type=pl.DeviceIdType.MESH,
        )

        @pl.when(d != my)
        def _(desc=desc):
            desc.wait_recv()

    for t in range(N_DEV):
        @pl.when(t != my)
        def _(s1=rs_sends[t], s2=ag_sends[t]):
            s1.wait_send()
            s2.wait_send()


def kernel(x, Wq, K_ext, V_ext, Wo):
    i = lax.axis_index("i")
    wq_i = lax.dynamic_slice(Wq, (0, i * HQ * DH), (D_MODEL, HQ * DH))
    wo_i = lax.dynamic_slice(Wo, (i * HQ * DH, 0), (HQ * DH, D_MODEL))

    xb = x[0].astype(jnp.bfloat16)
    wq_b = wq_i.astype(jnp.bfloat16)
    wo_b = wo_i.astype(jnp.bfloat16)
    kb = jnp.transpose(K_ext[0], (1, 0, 2)).astype(jnp.bfloat16)
    vb = jnp.transpose(V_ext[0], (1, 0, 2)).astype(jnp.bfloat16)

    out = pl.pallas_call(
        _body,
        out_shape=jax.ShapeDtypeStruct((SQ, D_MODEL), jnp.bfloat16),
        in_specs=[pl.BlockSpec(memory_space=pltpu.VMEM)] * 5,
        out_specs=pl.BlockSpec(memory_space=pltpu.VMEM),
        scratch_shapes=[
            pltpu.VMEM((SQ, HQ * DH), jnp.bfloat16),
            pltpu.VMEM((SQ, D_MODEL), jnp.bfloat16),
            pltpu.VMEM((N_DEV, CHUNK, D_MODEL), jnp.bfloat16),
            pltpu.SemaphoreType.DMA((N_DEV,)),
            pltpu.SemaphoreType.DMA((N_DEV,)),
            pltpu.SemaphoreType.DMA((N_DEV,)),
            pltpu.SemaphoreType.DMA((N_DEV,)),
        ],
    )(xb, wq_b, kb, vb, wo_b)
    return out.astype(jnp.float32)[None]


# device time: 91493 ns/iter; 2.7261x vs baseline; 1.1141x over previous
import jax
import jax.numpy as jnp
from jax import lax
from jax.experimental import pallas as pl
from jax.experimental.pallas import tpu as pltpu

N_DEV = 32
SQ = 1024
SKV = 1024
HQ = 8
DH = 128
D_MODEL = 1024
WINDOW = 128
SCALE = 0.08838834764831843
CHUNK = SQ // N_DEV
RB = 256
CW = 512


def _body(x_ref, wq_ref, k_ref, v_ref, wo_ref, out_ref,
          ctx_ref, part_ref, rs_buf,
          rs_send_sems, rs_recv_sems, ag_send_sems, ag_recv_sems):
    my = lax.axis_index("i")

    q = jnp.dot(x_ref[:], wq_ref[:], preferred_element_type=jnp.float32)

    rs_sends = [None] * N_DEV
    for rb in range(SQ // RB):
        r0 = rb * RB
        cs = min(max(r0 - WINDOW, 0), SKV - CW)
        qi = lax.broadcasted_iota(jnp.int32, (RB, CW), 0) + r0
        ki = lax.broadcasted_iota(jnp.int32, (RB, CW), 1) + cs
        bias = jnp.where(jnp.abs(qi - ki) <= WINDOW,
                         0.0, -1e9).astype(jnp.float32)

        for h in range(HQ):
            qh = q[r0:r0 + RB, h * DH:(h + 1) * DH].astype(jnp.bfloat16)
            kh = k_ref[h, cs:cs + CW, :]
            vh = v_ref[h, cs:cs + CW, :]
            s = lax.dot_general(qh, kh, (((1,), (1,)), ((), ())),
                                preferred_element_type=jnp.float32)
            s = s * SCALE + bias
            m = jnp.max(s, axis=1, keepdims=True)
            w = jnp.exp(s - m)
            w = w / jnp.sum(w, axis=1, keepdims=True)
            ctx_ref[r0:r0 + RB, h * DH:(h + 1) * DH] = jnp.dot(
                w.astype(jnp.bfloat16), vh,
                preferred_element_type=jnp.float32).astype(jnp.bfloat16)

        part_ref[r0:r0 + RB, :] = jnp.dot(
            ctx_ref[r0:r0 + RB, :], wo_ref[:],
            preferred_element_type=jnp.float32).astype(jnp.bfloat16)

        for j in range(RB // CHUNK):
            t = rb * (RB // CHUNK) + j
            desc = pltpu.make_async_remote_copy(
                src_ref=part_ref.at[pl.ds(t * CHUNK, CHUNK), :],
                dst_ref=rs_buf.at[my],
                send_sem=rs_send_sems.at[t],
                recv_sem=rs_recv_sems.at[my],
                device_id=(t,),
                device_id_type=pl.DeviceIdType.MESH,
            )
            rs_sends[t] = desc

            @pl.when(t != my)
            def _(desc=desc):
                desc.start()

    rs_buf[my, :, :] = part_ref[pl.ds(my * CHUNK, CHUNK), :]

    for d in range(N_DEV):
        desc = pltpu.make_async_remote_copy(
            src_ref=part_ref.at[pl.ds(0, CHUNK), :],
            dst_ref=rs_buf.at[d],
            send_sem=rs_send_sems.at[d],
            recv_sem=rs_recv_sems.at[d],
            device_id=(0,),
            device_id_type=pl.DeviceIdType.MESH,
        )

        @pl.when(d != my)
        def _(desc=desc):
            desc.wait_recv()

    acc = jnp.sum(rs_buf[:].astype(jnp.float32), axis=0)
    my_rows = pl.ds(my * CHUNK, CHUNK)
    out_ref[my_rows, :] = acc.astype(jnp.bfloat16)

    ag_sends = []
    for t in range(N_DEV):
        desc = pltpu.make_async_remote_copy(
            src_ref=out_ref.at[my_rows, :],
            dst_ref=out_ref.at[my_rows, :],
            send_sem=ag_send_sems.at[t],
            recv_sem=ag_recv_sems.at[my],
            device_id=(t,),
            device_id_type=pl.DeviceIdType.MESH,
        )
        ag_sends.append(desc)

        @pl.when(t != my)
        def _(desc=desc):
            desc.start()

    for d in range(N_DEV):
        desc = pltpu.make_async_remote_copy(
            src_ref=out_ref.at[my_rows, :],
            dst_ref=out_ref.at[pl.ds(d * CHUNK, CHUNK), :],
            send_sem=ag_send_sems.at[d],
            recv_sem=ag_recv_sems.at[d],
            device_id=(0,),
            device_id_type=pl.DeviceIdType.MESH,
        )

        @pl.when(d != my)
        def _(desc=desc):
            desc.wait_recv()

    for t in range(N_DEV):
        @pl.when(t != my)
        def _(s1=rs_sends[t], s2=ag_sends[t]):
            s1.wait_send()
            s2.wait_send()


def kernel(x, Wq, K_ext, V_ext, Wo):
    i = lax.axis_index("i")
    wq_i = lax.dynamic_slice(Wq, (0, i * HQ * DH), (D_MODEL, HQ * DH))
    wo_i = lax.dynamic_slice(Wo, (i * HQ * DH, 0), (HQ * DH, D_MODEL))

    xb = x[0].astype(jnp.bfloat16)
    wq_b = wq_i.astype(jnp.bfloat16)
    wo_b = wo_i.astype(jnp.bfloat16)
    kb = jnp.transpose(K_ext[0], (1, 0, 2)).astype(jnp.bfloat16)
    vb = jnp.transpose(V_ext[0], (1, 0, 2)).astype(jnp.bfloat16)

    out = pl.pallas_call(
        _body,
        out_shape=jax.ShapeDtypeStruct((SQ, D_MODEL), jnp.bfloat16),
        in_specs=[pl.BlockSpec(memory_space=pltpu.VMEM)] * 5,
        out_specs=pl.BlockSpec(memory_space=pltpu.VMEM),
        scratch_shapes=[
            pltpu.VMEM((SQ, HQ * DH), jnp.bfloat16),
            pltpu.VMEM((SQ, D_MODEL), jnp.bfloat16),
            pltpu.VMEM((N_DEV, CHUNK, D_MODEL), jnp.bfloat16),
            pltpu.SemaphoreType.DMA((N_DEV,)),
            pltpu.SemaphoreType.DMA((N_DEV,)),
            pltpu.SemaphoreType.DMA((N_DEV,)),
            pltpu.SemaphoreType.DMA((N_DEV,)),
        ],
    )(xb, wq_b, kb, vb, wo_b)
    return out.astype(jnp.float32)[None]
